# baseline (speedup 1.0000x reference)
import functools

import jax
import jax.numpy as jnp
from jax import lax
from jax.experimental import pallas as pl
from jax.experimental.pallas import tpu as pltpu
from jax.experimental.pallas import tpu_sc as plsc

_VOCAB = 1024
_EMBED = 2048
_OUT = 1024
_BATCH = 16384

_INFO = plsc.get_sparse_core_info()
_NC = _INFO.num_cores
_NS = _INFO.num_subcores
_NW = _NC * _NS
_B_PER_W = _BATCH // _NW
_CHUNK = 32
_N_CHUNKS = _B_PER_W // _CHUNK


def _mm_body(t_ref, w_ref, o_ref):
    o_ref[...] = lax.dot_general(
        t_ref[...], w_ref[...],
        dimension_numbers=(((1,), (1,)), ((), ())),
        preferred_element_type=jnp.float32,
    )


def _project_table(embed_table, w):
    return pl.pallas_call(
        _mm_body,
        out_shape=jax.ShapeDtypeStruct((_VOCAB, _OUT), jnp.float32),
    )(embed_table, w)


_sc_mesh = plsc.VectorSubcoreMesh(core_axis_name="c", subcore_axis_name="s")


_NBUF = 3


@functools.partial(
    pl.kernel,
    mesh=_sc_mesh,
    out_type=jax.ShapeDtypeStruct((_BATCH, _OUT), jnp.float32),
    scratch_types=[
        pltpu.VMEM((_N_CHUNKS, _CHUNK), jnp.int32),
    ]
    + [pltpu.VMEM((_CHUNK, _OUT), jnp.float32) for _ in range(_NBUF)]
    + [pltpu.SemaphoreType.DMA for _ in range(2 * _NBUF)],
)
def _sc_gather(p_hbm, idx_hbm, out_hbm, idx_v, *bufs_and_sems):
    bufs = bufs_and_sems[:_NBUF]
    gsems = bufs_and_sems[_NBUF:2 * _NBUF]
    wsems = bufs_and_sems[2 * _NBUF:]
    wid = lax.axis_index("s") * _NC + lax.axis_index("c")
    base = wid * _B_PER_W
    pltpu.sync_copy(idx_hbm.at[wid], idx_v)
    gathers = [None] * _N_CHUNKS
    writes = [None] * _N_CHUNKS
    for i in range(_NBUF):
        gathers[i] = pltpu.async_copy(p_hbm.at[idx_v.at[i]], bufs[i % _NBUF],
                                      gsems[i % _NBUF])
    for i in range(_N_CHUNKS):
        w = i - 1
        g = w + _NBUF
        if w >= 0 and g < _N_CHUNKS:
            writes[w].wait()
            gathers[g] = pltpu.async_copy(p_hbm.at[idx_v.at[g]], bufs[g % _NBUF],
                                          gsems[g % _NBUF])
        gathers[i].wait()
        writes[i] = pltpu.async_copy(bufs[i % _NBUF],
                                     out_hbm.at[pl.ds(base + i * _CHUNK, _CHUNK)],
                                     wsems[i % _NBUF])
    for w in range(max(0, _N_CHUNKS - _NBUF - 1), _N_CHUNKS):
        if writes[w] is not None and w >= _N_CHUNKS - _NBUF:
            writes[w].wait()


def kernel(x, embed_table, W):
    p = _project_table(embed_table, W)
    idx = x.astype(jnp.int32).reshape(_NW, _N_CHUNKS, _CHUNK)
    return _sc_gather(p, idx)


# speedup vs baseline: 1.0442x; 1.0442x over previous
import functools

import jax
import jax.numpy as jnp
from jax import lax
from jax.experimental import pallas as pl
from jax.experimental.pallas import tpu as pltpu
from jax.experimental.pallas import tpu_sc as plsc

_VOCAB = 1024
_EMBED = 2048
_OUT = 1024
_BATCH = 16384

_INFO = plsc.get_sparse_core_info()
_NC = _INFO.num_cores
_NS = _INFO.num_subcores
_NW = _NC * _NS
_B_PER_W = _BATCH // _NW
_CHUNK = 32
_N_CHUNKS = _B_PER_W // _CHUNK


def _mm_body(t_ref, w_ref, o_ref):
    o_ref[...] = lax.dot_general(
        t_ref[...], w_ref[...],
        dimension_numbers=(((1,), (1,)), ((), ())),
        precision=lax.Precision.DEFAULT,
        preferred_element_type=jnp.float32,
    )


def _project_table(embed_table, w):
    return pl.pallas_call(
        _mm_body,
        out_shape=jax.ShapeDtypeStruct((_VOCAB, _OUT), jnp.float32),
    )(embed_table, w)


_sc_mesh = plsc.VectorSubcoreMesh(core_axis_name="c", subcore_axis_name="s")


_NBUF = 3


@functools.partial(
    pl.kernel,
    mesh=_sc_mesh,
    out_type=jax.ShapeDtypeStruct((_BATCH, _OUT), jnp.float32),
    scratch_types=[
        pltpu.VMEM((_B_PER_W,), jnp.int32),
    ]
    + [pltpu.VMEM((_CHUNK, _OUT), jnp.float32) for _ in range(_NBUF)]
    + [pltpu.SemaphoreType.DMA for _ in range(2 * _NBUF)],
)
def _sc_gather(p_hbm, idx_hbm, out_hbm, idx_v, *bufs_and_sems):
    bufs = bufs_and_sems[:_NBUF]
    gsems = bufs_and_sems[_NBUF:2 * _NBUF]
    wsems = bufs_and_sems[2 * _NBUF:]
    wid = lax.axis_index("s") * _NC + lax.axis_index("c")
    base = wid * _B_PER_W
    pltpu.sync_copy(idx_hbm.at[pl.ds(base, _B_PER_W)], idx_v)
    gathers = [None] * _N_CHUNKS
    writes = [None] * _N_CHUNKS

    def chunk_idx(i):
        return idx_v.at[pl.ds(i * _CHUNK, _CHUNK)]

    for i in range(_NBUF):
        gathers[i] = pltpu.async_copy(p_hbm.at[chunk_idx(i)], bufs[i % _NBUF],
                                      gsems[i % _NBUF])
    for i in range(_N_CHUNKS):
        w = i - 1
        g = w + _NBUF
        if w >= 0 and g < _N_CHUNKS:
            writes[w].wait()
            gathers[g] = pltpu.async_copy(p_hbm.at[chunk_idx(g)], bufs[g % _NBUF],
                                          gsems[g % _NBUF])
        gathers[i].wait()
        writes[i] = pltpu.async_copy(bufs[i % _NBUF],
                                     out_hbm.at[pl.ds(base + i * _CHUNK, _CHUNK)],
                                     wsems[i % _NBUF])
    for w in range(max(0, _N_CHUNKS - _NBUF - 1), _N_CHUNKS):
        if writes[w] is not None and w >= _N_CHUNKS - _NBUF:
            writes[w].wait()


def kernel(x, embed_table, W):
    p = _project_table(embed_table, W)
    return _sc_gather(p, x.astype(jnp.int32))
